# baseline (speedup 1.0000x reference)
import functools

import jax
import jax.numpy as jnp
from jax import lax
from jax.experimental import pallas as pl
from jax.experimental.pallas import tpu as pltpu
from jax.experimental.pallas import tpu_sc as plsc

VOCAB = 50368
HIDDEN = 768
EPS = 1e-05

_NC = 2
_NS = 16
_NW = _NC * _NS
_CHUNK = 64


def _sc_gather(table, ids_flat):
    bs = ids_flat.shape[0]
    b_per_w = bs // _NW
    n_chunks = b_per_w // _CHUNK
    ids2 = ids_flat.reshape(bs // _CHUNK, _CHUNK)
    mesh = plsc.VectorSubcoreMesh(core_axis_name="c", subcore_axis_name="s")

    @functools.partial(
        pl.kernel,
        mesh=mesh,
        out_type=jax.ShapeDtypeStruct((bs, HIDDEN), jnp.float32),
        scratch_types=[
            pltpu.VMEM((n_chunks, _CHUNK), jnp.int32),
            pltpu.VMEM((_CHUNK, HIDDEN), jnp.float32),
            pltpu.VMEM((_CHUNK, HIDDEN), jnp.float32),
            pltpu.SemaphoreType.DMA,
            pltpu.SemaphoreType.DMA,
        ],
    )
    def gather_kernel(table_hbm, ids_hbm, out_hbm, idx_v, rows_a, rows_b, sem_g, sem_w):
        wid = lax.axis_index("s") * _NC + lax.axis_index("c")
        base = wid * b_per_w
        pltpu.sync_copy(ids_hbm.at[pl.ds(wid * n_chunks, n_chunks)], idx_v)
        bufs = (rows_a, rows_b)
        writes = [None] * n_chunks
        for ci in range(n_chunks):
            buf = bufs[ci % 2]
            if ci >= 2:
                writes[ci - 2].wait()
            pltpu.async_copy(table_hbm.at[idx_v.at[ci]], buf, sem_g).wait()
            writes[ci] = pltpu.async_copy(
                buf, out_hbm.at[pl.ds(base + ci * _CHUNK, _CHUNK)], sem_w)
        writes[n_chunks - 2].wait()
        writes[n_chunks - 1].wait()

    return gather_kernel(table, ids2)


def _ln_transpose_body(rows_ref, w_ref, out_ref):
    x = rows_ref[0]
    t = x.shape[0]
    mean = jnp.mean(x, axis=1, keepdims=True)
    zm = x - mean
    var = jnp.mean(zm * zm, axis=1, keepdims=True)
    y = zm * lax.rsqrt(var + EPS) * w_ref[...]
    out_ref[0, :, 0, :] = y.T


def _tc_ln_transpose(rows, ln_weight, b, s):
    t = 2048
    grid = (b, s // t)
    rows3 = rows.reshape(b, s, HIDDEN)
    return pl.pallas_call(
        _ln_transpose_body,
        grid=grid,
        in_specs=[
            pl.BlockSpec((1, t, HIDDEN), lambda i, j: (i, j, 0)),
            pl.BlockSpec((1, HIDDEN), lambda i, j: (0, 0)),
        ],
        out_specs=pl.BlockSpec((1, HIDDEN, 1, t), lambda i, j: (i, 0, 0, j)),
        out_shape=jax.ShapeDtypeStruct((b, HIDDEN, 1, s), jnp.float32),
    )(rows3, ln_weight.reshape(1, HIDDEN))


def kernel(input_ids, table, ln_weight):
    b, s = input_ids.shape
    ids_flat = input_ids.reshape(b * s).astype(jnp.int32)
    rows = _sc_gather(table, ids_flat)
    return _tc_ln_transpose(rows, ln_weight, b, s)


# speedup vs baseline: 1.0036x; 1.0036x over previous
import functools

import jax
import jax.numpy as jnp
from jax import lax
from jax.experimental import pallas as pl
from jax.experimental.pallas import tpu as pltpu
from jax.experimental.pallas import tpu_sc as plsc

VOCAB = 50368
HIDDEN = 768
EPS = 1e-05

_NC = 2
_NS = 16
_NW = _NC * _NS
_CHUNK = 64


def _sc_gather(table, ids_flat):
    bs = ids_flat.shape[0]
    b_per_w = bs // _NW
    n_chunks = b_per_w // _CHUNK
    ids2 = ids_flat.reshape(bs // _CHUNK, _CHUNK)
    mesh = plsc.VectorSubcoreMesh(core_axis_name="c", subcore_axis_name="s")

    @functools.partial(
        pl.kernel,
        mesh=mesh,
        out_type=jax.ShapeDtypeStruct((bs, HIDDEN), jnp.float32),
        scratch_types=[
            pltpu.VMEM((n_chunks, _CHUNK), jnp.int32),
            pltpu.VMEM((_CHUNK, HIDDEN), jnp.float32),
            pltpu.VMEM((_CHUNK, HIDDEN), jnp.float32),
            pltpu.SemaphoreType.DMA,
            pltpu.SemaphoreType.DMA,
        ],
    )
    def gather_kernel(table_hbm, ids_hbm, out_hbm, idx_v, rows_a, rows_b, sem_g, sem_w):
        wid = lax.axis_index("s") * _NC + lax.axis_index("c")
        base = wid * b_per_w
        pltpu.sync_copy(ids_hbm.at[pl.ds(wid * n_chunks, n_chunks)], idx_v)
        bufs = (rows_a, rows_b)
        writes = [None] * n_chunks
        for ci in range(n_chunks):
            buf = bufs[ci % 2]
            if ci >= 2:
                writes[ci - 2].wait()
            pltpu.async_copy(table_hbm.at[idx_v.at[ci]], buf, sem_g).wait()
            writes[ci] = pltpu.async_copy(
                buf, out_hbm.at[pl.ds(base + ci * _CHUNK, _CHUNK)], sem_w)
        writes[n_chunks - 2].wait()
        writes[n_chunks - 1].wait()

    return gather_kernel(table, ids2)


def _ln_body(rows_ref, w_ref, out_ref):
    x = rows_ref[...]
    mean = jnp.mean(x, axis=1, keepdims=True)
    zm = x - mean
    var = jnp.mean(zm * zm, axis=1, keepdims=True)
    y = zm * lax.rsqrt(var + EPS) * w_ref[...]
    out_ref[0, :, 0, :] = y.T


def _ln_body_alias(rows_ref, w_ref, prev_ref, out_ref):
    del prev_ref
    _ln_body(rows_ref, w_ref, out_ref)


def _tc_ln_chunk(rows, w2, out_prev, bi, b, s):
    t = 2048
    grid = (s // t,)
    in_specs = [
        pl.BlockSpec((t, HIDDEN), lambda j: (j, 0)),
        pl.BlockSpec((1, HIDDEN), lambda j: (0, 0)),
    ]
    args = [rows, w2]
    kwargs = {}
    body = _ln_body
    if out_prev is not None:
        in_specs.append(pl.BlockSpec(memory_space=pl.ANY))
        args.append(out_prev)
        kwargs["input_output_aliases"] = {2: 0}
        body = _ln_body_alias
    return pl.pallas_call(
        body,
        grid=grid,
        in_specs=in_specs,
        out_specs=pl.BlockSpec((1, HIDDEN, 1, t), lambda j: (bi, 0, 0, j)),
        out_shape=jax.ShapeDtypeStruct((b, HIDDEN, 1, s), jnp.float32),
        **kwargs,
    )(*args)


def kernel(input_ids, table, ln_weight):
    b, s = input_ids.shape
    w2 = ln_weight.reshape(1, HIDDEN)
    ids = input_ids.astype(jnp.int32)
    out = None
    for bi in range(b):
        rows = _sc_gather(table, ids[bi])
        out = _tc_ln_chunk(rows, w2, out, bi, b, s)
    return out


# speedup vs baseline: 1.0057x; 1.0021x over previous
import functools

import jax
import jax.numpy as jnp
from jax import lax
from jax.experimental import pallas as pl
from jax.experimental.pallas import tpu as pltpu
from jax.experimental.pallas import tpu_sc as plsc

VOCAB = 50368
HIDDEN = 768
EPS = 1e-05

_NC = 2
_NS = 16
_NW = _NC * _NS
_CHUNK = 128


def _sc_gather(table, ids_flat):
    bs = ids_flat.shape[0]
    b_per_w = bs // _NW
    n_chunks = b_per_w // _CHUNK
    ids2 = ids_flat.reshape(bs // _CHUNK, _CHUNK)
    mesh = plsc.VectorSubcoreMesh(core_axis_name="c", subcore_axis_name="s")

    @functools.partial(
        pl.kernel,
        mesh=mesh,
        out_type=jax.ShapeDtypeStruct((bs, HIDDEN), jnp.float32),
        scratch_types=[
            pltpu.VMEM((n_chunks, _CHUNK), jnp.int32),
            pltpu.VMEM((_CHUNK, HIDDEN), jnp.float32),
            pltpu.SemaphoreType.DMA,
        ],
    )
    def gather_kernel(table_hbm, ids_hbm, out_hbm, idx_v, rows_v, sem_g):
        wid = lax.axis_index("s") * _NC + lax.axis_index("c")
        base = wid * b_per_w
        pltpu.sync_copy(ids_hbm.at[pl.ds(wid * n_chunks, n_chunks)], idx_v)
        for ci in range(n_chunks):
            pltpu.async_copy(table_hbm.at[idx_v.at[ci]], rows_v, sem_g).wait()
            pltpu.sync_copy(rows_v, out_hbm.at[pl.ds(base + ci * _CHUNK, _CHUNK)])

    return gather_kernel(table, ids2)


def _ln_body(rows_ref, w_ref, out_ref):
    x = rows_ref[...]
    mean = jnp.mean(x, axis=1, keepdims=True)
    zm = x - mean
    var = jnp.mean(zm * zm, axis=1, keepdims=True)
    y = zm * lax.rsqrt(var + EPS) * w_ref[...]
    out_ref[0, :, 0, :] = y.T


def _ln_body_alias(rows_ref, w_ref, prev_ref, out_ref):
    del prev_ref
    _ln_body(rows_ref, w_ref, out_ref)


def _tc_ln_chunk(rows, w2, out_prev, bi, b, s):
    t = 2048
    grid = (s // t,)
    in_specs = [
        pl.BlockSpec((t, HIDDEN), lambda j: (j, 0)),
        pl.BlockSpec((1, HIDDEN), lambda j: (0, 0)),
    ]
    args = [rows, w2]
    kwargs = {}
    body = _ln_body
    if out_prev is not None:
        in_specs.append(pl.BlockSpec(memory_space=pl.ANY))
        args.append(out_prev)
        kwargs["input_output_aliases"] = {2: 0}
        body = _ln_body_alias
    return pl.pallas_call(
        body,
        grid=grid,
        in_specs=in_specs,
        out_specs=pl.BlockSpec((1, HIDDEN, 1, t), lambda j: (bi, 0, 0, j)),
        out_shape=jax.ShapeDtypeStruct((b, HIDDEN, 1, s), jnp.float32),
        **kwargs,
    )(*args)


def kernel(input_ids, table, ln_weight):
    b, s = input_ids.shape
    w2 = ln_weight.reshape(1, HIDDEN)
    ids = input_ids.astype(jnp.int32)
    out = None
    for bi in range(b):
        rows = _sc_gather(table, ids[bi])
        out = _tc_ln_chunk(rows, w2, out, bi, b, s)
    return out
